# baseline (speedup 1.0000x reference)
import functools

import jax
import jax.numpy as jnp
from jax import lax
from jax.experimental import pallas as pl
from jax.experimental.pallas import tpu as pltpu
from jax.experimental.pallas import tpu_sc as plsc

_N = 10000
_NPAD = 10240
_D = 128
_NC = 2
_NS = 16
_NT = _NC * _NS
_KB = 128
_NBATCH = 80
_EPW = _KB * _NBATCH
_EPAD = _EPW * _NT
_RPT = _NPAD // _NS
_BR = 1024

_sc_mesh = plsc.VectorSubcoreMesh(core_axis_name="c", subcore_axis_name="s")


def _fill_rows(rows_v, val):
    def _f(i, c):
        rows_v[i // (_D // 16), pl.ds((i % (_D // 16)) * 16, 16)] = jnp.full(
            (16,), val, jnp.float32)
        return c
    lax.fori_loop(0, _KB * (_D // 16), _f, 0)


@functools.partial(
    pl.kernel,
    out_type=jax.ShapeDtypeStruct((_NC, _NPAD, _D), jnp.float32),
    mesh=_sc_mesh,
    scratch_types=[
        pltpu.VMEM((_NBATCH, _KB), jnp.int32),
        pltpu.VMEM((_KB, _D), jnp.float32),
        pltpu.VMEM_SHARED((_NPAD, _D), jnp.float32),
        pltpu.SemaphoreType.DMA,
    ],
)
def _deg_kernel(dst_hbm, deg_hbm, didx, rows_v, deg_s, sem):
    core = lax.axis_index("c")
    sub = lax.axis_index("s")
    tid = sub * _NC + core

    _fill_rows(rows_v, 0.0)

    def _zero(r, c):
        pltpu.sync_copy(rows_v, deg_s.at[pl.ds(sub * _RPT + r * _KB, _KB), :])
        return c

    lax.fori_loop(0, _RPT // _KB, _zero, 0)
    _fill_rows(rows_v, 1.0)
    pltpu.sync_copy(dst_hbm.at[pl.ds(tid * _NBATCH, _NBATCH), :], didx)
    plsc.subcore_barrier()

    def _fire(i, c):
        pltpu.async_copy(rows_v, deg_s.at[didx.at[2 * i]], sem, add=True)
        pltpu.async_copy(rows_v, deg_s.at[didx.at[2 * i + 1]], sem, add=True)
        pltpu.make_async_copy(rows_v, deg_s.at[didx.at[2 * i]], sem).wait()
        pltpu.make_async_copy(rows_v, deg_s.at[didx.at[2 * i + 1]], sem).wait()
        return c

    lax.fori_loop(0, _NBATCH // 2, _fire, 0)
    plsc.subcore_barrier()

    pltpu.sync_copy(
        deg_s.at[pl.ds(sub * _RPT, _RPT), :],
        deg_hbm.at[core, pl.ds(sub * _RPT, _RPT), :],
    )


_NB0 = 96
_NB1 = 64


@functools.partial(
    pl.kernel,
    out_type=jax.ShapeDtypeStruct((_NC, _NPAD, _D), jnp.float32),
    mesh=_sc_mesh,
    scratch_types=[
        pltpu.VMEM((_NB0 // 4, _KB), jnp.int32),
        pltpu.VMEM((_NB0 // 4, _KB), jnp.int32),
        pltpu.VMEM((_KB, _D), jnp.float32),
        pltpu.VMEM((_KB, _D), jnp.float32),
        pltpu.VMEM_SHARED((_NPAD, _D), jnp.float32),
        pltpu.SemaphoreType.DMA,
        pltpu.SemaphoreType.DMA,
        pltpu.SemaphoreType.DMA,
        pltpu.SemaphoreType.DMA,
    ],
)
def _scatter_kernel(g_hbm, src_hbm, dst_hbm, agg_hbm, sidx, didx,
                    r0, r1, agg_s, g0, g1, s0, s1):
    core = lax.axis_index("c")
    sub = lax.axis_index("s")

    def gstart(i, buf, sem):
        pltpu.async_copy(g_hbm.at[sidx.at[i]], buf, sem)

    def gwait(i, buf, sem):
        pltpu.make_async_copy(g_hbm.at[sidx.at[i]], buf, sem).wait()

    def sstart(i, buf, sem):
        pltpu.async_copy(buf, agg_s.at[didx.at[i]], sem, add=True)

    def swait(i, buf, sem):
        pltpu.make_async_copy(buf, agg_s.at[didx.at[i]], sem).wait()

    _fill_rows(r0, 0.0)

    def _zero(r, c):
        pltpu.sync_copy(r0, agg_s.at[pl.ds(sub * _RPT + r * _KB, _KB), :])
        return c

    lax.fori_loop(0, _RPT // _KB, _zero, 0)
    plsc.subcore_barrier()

    def _run_edges(nb, rowbase):
        nh = nb // 4
        for h in range(4):
            base = rowbase + h * nh
            pltpu.sync_copy(src_hbm.at[pl.ds(base, nh), :],
                            sidx.at[pl.ds(0, nh), :])
            pltpu.sync_copy(dst_hbm.at[pl.ds(base, nh), :],
                            didx.at[pl.ds(0, nh), :])
            gstart(0, r0, g0)
            gstart(1, r1, g1)

            def _body(t, c):
                j0 = 2 * t
                j1 = 2 * t + 1
                gwait(j0, r0, g0)
                sstart(j0, r0, s0)
                gwait(j1, r1, g1)
                sstart(j1, r1, s1)
                swait(j0, r0, s0)
                gstart(j0 + 2, r0, g0)
                swait(j1, r1, s1)
                gstart(j1 + 2, r1, g1)
                return c

            lax.fori_loop(0, nh // 2 - 1, _body, 0)
            gwait(nh - 2, r0, g0)
            sstart(nh - 2, r0, s0)
            gwait(nh - 1, r1, g1)
            sstart(nh - 1, r1, s1)
            swait(nh - 2, r0, s0)
            swait(nh - 1, r1, s1)

    @pl.when(core == 1)
    def _():
        _run_edges(_NB0, sub * _NB0)

    @pl.when(core == 0)
    def _():
        _run_edges(_NB1, _NS * _NB0 + sub * _NB1)

    plsc.subcore_barrier()

    pltpu.sync_copy(
        agg_s.at[pl.ds(sub * _RPT, _RPT), :],
        agg_hbm.at[core, pl.ds(sub * _RPT, _RPT), :],
    )


def _dot(a, b):
    return jnp.dot(a, b, preferred_element_type=jnp.float32,
                   precision=lax.Precision.HIGHEST)


def _pre_body(deg_ref, x_ref, w_ref, dis_ref, g_ref):
    d = deg_ref[0, :, 0:1] + deg_ref[1, :, 0:1] + 1.0
    dis = lax.rsqrt(d)
    dis_ref[...] = dis
    g_ref[...] = _dot(x_ref[...], w_ref[...]) * dis


_pre_call = pl.pallas_call(
    _pre_body,
    grid=(_NPAD // _BR,),
    in_specs=[
        pl.BlockSpec((_NC, _BR, _D), lambda i: (0, i, 0)),
        pl.BlockSpec((_BR, _D), lambda i: (i, 0)),
        pl.BlockSpec((_D, _D), lambda i: (0, 0)),
    ],
    out_specs=[
        pl.BlockSpec((_BR, 1), lambda i: (i, 0)),
        pl.BlockSpec((_BR, _D), lambda i: (i, 0)),
    ],
    out_shape=[
        jax.ShapeDtypeStruct((_NPAD, 1), jnp.float32),
        jax.ShapeDtypeStruct((_NPAD, _D), jnp.float32),
    ],
)


def _mid_body(agg_ref, g_ref, dis_ref, b_ref, w_ref, out_ref):
    dis = dis_ref[...]
    a = agg_ref[0] + agg_ref[1] + g_ref[...]
    h = jnp.maximum(dis * a + b_ref[...], 0.0)
    out_ref[...] = _dot(h, w_ref[...]) * dis


_mid_call = pl.pallas_call(
    _mid_body,
    grid=(_NPAD // _BR,),
    in_specs=[
        pl.BlockSpec((_NC, _BR, _D), lambda i: (0, i, 0)),
        pl.BlockSpec((_BR, _D), lambda i: (i, 0)),
        pl.BlockSpec((_BR, 1), lambda i: (i, 0)),
        pl.BlockSpec((1, _D), lambda i: (0, 0)),
        pl.BlockSpec((_D, _D), lambda i: (0, 0)),
    ],
    out_specs=pl.BlockSpec((_BR, _D), lambda i: (i, 0)),
    out_shape=jax.ShapeDtypeStruct((_NPAD, _D), jnp.float32),
)


def _fin_body(agg_ref, g_ref, dis_ref, b_ref, wl_ref, bl_ref, out_ref):
    dis = dis_ref[...]
    a = agg_ref[0] + agg_ref[1] + g_ref[...]
    h = jnp.maximum(dis * a + b_ref[...], 0.0)
    out_ref[...] = _dot(h, wl_ref[...]) + bl_ref[...]


def _make_fin(n_classes):
    return pl.pallas_call(
        _fin_body,
        grid=(_NPAD // _BR,),
        in_specs=[
            pl.BlockSpec((_NC, _BR, _D), lambda i: (0, i, 0)),
            pl.BlockSpec((_BR, _D), lambda i: (i, 0)),
            pl.BlockSpec((_BR, 1), lambda i: (i, 0)),
            pl.BlockSpec((1, _D), lambda i: (0, 0)),
            pl.BlockSpec((_D, n_classes), lambda i: (0, 0)),
            pl.BlockSpec((1, n_classes), lambda i: (0, 0)),
        ],
        out_specs=pl.BlockSpec((_BR, n_classes), lambda i: (i, 0)),
        out_shape=jax.ShapeDtypeStruct((_NPAD, n_classes), jnp.float32),
    )


def kernel(x, edge_index, W1, b1, W2, b2, W3, b3, Wl, bl):
    src = edge_index[0].astype(jnp.int32)
    dst = edge_index[1].astype(jnp.int32)
    pad_e = _EPAD - src.shape[0]
    src = jnp.pad(src, (0, pad_e), constant_values=_N).reshape(-1, _KB)
    dst = jnp.pad(dst, (0, pad_e), constant_values=_N).reshape(-1, _KB)
    xp = jnp.pad(x, ((0, _NPAD - x.shape[0]), (0, 0)))

    deg = _deg_kernel(dst)
    dis, g1 = _pre_call(deg, xp, W1)
    agg1 = _scatter_kernel(g1, src, dst)
    g2 = _mid_call(agg1, g1, dis, b1.reshape(1, -1), W2)
    agg2 = _scatter_kernel(g2, src, dst)
    g3 = _mid_call(agg2, g2, dis, b2.reshape(1, -1), W3)
    agg3 = _scatter_kernel(g3, src, dst)
    out = _make_fin(Wl.shape[1])(agg3, g3, dis, b3.reshape(1, -1), Wl,
                                 bl.reshape(1, -1))
    return out[:x.shape[0]]


# speedup vs baseline: 1.0578x; 1.0578x over previous
import functools

import jax
import jax.numpy as jnp
from jax import lax
from jax.experimental import pallas as pl
from jax.experimental.pallas import tpu as pltpu
from jax.experimental.pallas import tpu_sc as plsc

_N = 10000
_NPAD = 10240
_D = 128
_NC = 2
_NS = 16
_NT = _NC * _NS
_KB = 128
_NBATCH = 80
_EPW = _KB * _NBATCH
_EPAD = _EPW * _NT
_RPT = _NPAD // _NS
_BR = 1024

_sc_mesh = plsc.VectorSubcoreMesh(core_axis_name="c", subcore_axis_name="s")


def _fill_rows(rows_v, val):
    def _f(i, c):
        rows_v[i // (_D // 16), pl.ds((i % (_D // 16)) * 16, 16)] = jnp.full(
            (16,), val, jnp.float32)
        return c
    lax.fori_loop(0, _KB * (_D // 16), _f, 0)


@functools.partial(
    pl.kernel,
    out_type=jax.ShapeDtypeStruct((_NC, _NPAD, _D), jnp.float32),
    mesh=_sc_mesh,
    scratch_types=[
        pltpu.VMEM((_NBATCH, _KB), jnp.int32),
        pltpu.VMEM((_KB, _D), jnp.float32),
        pltpu.VMEM_SHARED((_NPAD, _D), jnp.float32),
        pltpu.SemaphoreType.DMA,
    ],
)
def _deg_kernel(dst_hbm, deg_hbm, didx, rows_v, deg_s, sem):
    core = lax.axis_index("c")
    sub = lax.axis_index("s")
    tid = sub * _NC + core

    _fill_rows(rows_v, 0.0)

    def _zero(r, c):
        pltpu.sync_copy(rows_v, deg_s.at[pl.ds(sub * _RPT + r * _KB, _KB), :])
        return c

    lax.fori_loop(0, _RPT // _KB, _zero, 0)
    _fill_rows(rows_v, 1.0)
    pltpu.sync_copy(dst_hbm.at[pl.ds(tid * _NBATCH, _NBATCH), :], didx)
    plsc.subcore_barrier()

    def _fire(i, c):
        pltpu.async_copy(rows_v, deg_s.at[didx.at[2 * i]], sem, add=True)
        pltpu.async_copy(rows_v, deg_s.at[didx.at[2 * i + 1]], sem, add=True)
        pltpu.make_async_copy(rows_v, deg_s.at[didx.at[2 * i]], sem).wait()
        pltpu.make_async_copy(rows_v, deg_s.at[didx.at[2 * i + 1]], sem).wait()
        return c

    lax.fori_loop(0, _NBATCH // 2, _fire, 0)
    plsc.subcore_barrier()

    pltpu.sync_copy(
        deg_s.at[pl.ds(sub * _RPT, _RPT), :],
        deg_hbm.at[core, pl.ds(sub * _RPT, _RPT), :],
    )


_NB0 = 128
_NB1 = 32


@functools.partial(
    pl.kernel,
    out_type=jax.ShapeDtypeStruct((_NC, _NPAD, _D), jnp.float32),
    mesh=_sc_mesh,
    scratch_types=[
        pltpu.VMEM((_NB0 // 4, _KB), jnp.int32),
        pltpu.VMEM((_NB0 // 4, _KB), jnp.int32),
        pltpu.VMEM((_KB, _D), jnp.float32),
        pltpu.VMEM((_KB, _D), jnp.float32),
        pltpu.VMEM_SHARED((_NPAD, _D), jnp.float32),
        pltpu.SemaphoreType.DMA,
        pltpu.SemaphoreType.DMA,
        pltpu.SemaphoreType.DMA,
        pltpu.SemaphoreType.DMA,
    ],
)
def _scatter_kernel(g_hbm, src_hbm, dst_hbm, agg_hbm, sidx, didx,
                    r0, r1, agg_s, g0, g1, s0, s1):
    core = lax.axis_index("c")
    sub = lax.axis_index("s")

    def gstart(i, buf, sem):
        pltpu.async_copy(g_hbm.at[sidx.at[i]], buf, sem)

    def gwait(i, buf, sem):
        pltpu.make_async_copy(g_hbm.at[sidx.at[i]], buf, sem).wait()

    def sstart(i, buf, sem):
        pltpu.async_copy(buf, agg_s.at[didx.at[i]], sem, add=True)

    def swait(i, buf, sem):
        pltpu.make_async_copy(buf, agg_s.at[didx.at[i]], sem).wait()

    _fill_rows(r0, 0.0)

    def _zero(r, c):
        pltpu.sync_copy(r0, agg_s.at[pl.ds(sub * _RPT + r * _KB, _KB), :])
        return c

    lax.fori_loop(0, _RPT // _KB, _zero, 0)
    plsc.subcore_barrier()

    def _run_edges(nb, rowbase):
        nh = nb // 4
        for h in range(4):
            base = rowbase + h * nh
            pltpu.sync_copy(src_hbm.at[pl.ds(base, nh), :],
                            sidx.at[pl.ds(0, nh), :])
            pltpu.sync_copy(dst_hbm.at[pl.ds(base, nh), :],
                            didx.at[pl.ds(0, nh), :])
            gstart(0, r0, g0)
            gstart(1, r1, g1)

            def _body(t, c):
                j0 = 2 * t
                j1 = 2 * t + 1
                gwait(j0, r0, g0)
                sstart(j0, r0, s0)
                gwait(j1, r1, g1)
                sstart(j1, r1, s1)
                swait(j0, r0, s0)
                gstart(j0 + 2, r0, g0)
                swait(j1, r1, s1)
                gstart(j1 + 2, r1, g1)
                return c

            lax.fori_loop(0, nh // 2 - 1, _body, 0)
            gwait(nh - 2, r0, g0)
            sstart(nh - 2, r0, s0)
            gwait(nh - 1, r1, g1)
            sstart(nh - 1, r1, s1)
            swait(nh - 2, r0, s0)
            swait(nh - 1, r1, s1)

    @pl.when(core == 1)
    def _():
        _run_edges(_NB0, sub * _NB0)

    @pl.when(core == 0)
    def _():
        _run_edges(_NB1, _NS * _NB0 + sub * _NB1)

    plsc.subcore_barrier()

    pltpu.sync_copy(
        agg_s.at[pl.ds(sub * _RPT, _RPT), :],
        agg_hbm.at[core, pl.ds(sub * _RPT, _RPT), :],
    )


def _dot(a, b):
    return jnp.dot(a, b, preferred_element_type=jnp.float32,
                   precision=lax.Precision.HIGHEST)


def _pre_body(deg_ref, x_ref, w_ref, dis_ref, g_ref):
    d = deg_ref[0, :, 0:1] + deg_ref[1, :, 0:1] + 1.0
    dis = lax.rsqrt(d)
    dis_ref[...] = dis
    g_ref[...] = _dot(x_ref[...], w_ref[...]) * dis


_pre_call = pl.pallas_call(
    _pre_body,
    grid=(_NPAD // _BR,),
    in_specs=[
        pl.BlockSpec((_NC, _BR, _D), lambda i: (0, i, 0)),
        pl.BlockSpec((_BR, _D), lambda i: (i, 0)),
        pl.BlockSpec((_D, _D), lambda i: (0, 0)),
    ],
    out_specs=[
        pl.BlockSpec((_BR, 1), lambda i: (i, 0)),
        pl.BlockSpec((_BR, _D), lambda i: (i, 0)),
    ],
    out_shape=[
        jax.ShapeDtypeStruct((_NPAD, 1), jnp.float32),
        jax.ShapeDtypeStruct((_NPAD, _D), jnp.float32),
    ],
)


def _mid_body(agg_ref, g_ref, dis_ref, b_ref, w_ref, out_ref):
    dis = dis_ref[...]
    a = agg_ref[0] + agg_ref[1] + g_ref[...]
    h = jnp.maximum(dis * a + b_ref[...], 0.0)
    out_ref[...] = _dot(h, w_ref[...]) * dis


_mid_call = pl.pallas_call(
    _mid_body,
    grid=(_NPAD // _BR,),
    in_specs=[
        pl.BlockSpec((_NC, _BR, _D), lambda i: (0, i, 0)),
        pl.BlockSpec((_BR, _D), lambda i: (i, 0)),
        pl.BlockSpec((_BR, 1), lambda i: (i, 0)),
        pl.BlockSpec((1, _D), lambda i: (0, 0)),
        pl.BlockSpec((_D, _D), lambda i: (0, 0)),
    ],
    out_specs=pl.BlockSpec((_BR, _D), lambda i: (i, 0)),
    out_shape=jax.ShapeDtypeStruct((_NPAD, _D), jnp.float32),
)


def _fin_body(agg_ref, g_ref, dis_ref, b_ref, wl_ref, bl_ref, out_ref):
    dis = dis_ref[...]
    a = agg_ref[0] + agg_ref[1] + g_ref[...]
    h = jnp.maximum(dis * a + b_ref[...], 0.0)
    out_ref[...] = _dot(h, wl_ref[...]) + bl_ref[...]


def _make_fin(n_classes):
    return pl.pallas_call(
        _fin_body,
        grid=(_NPAD // _BR,),
        in_specs=[
            pl.BlockSpec((_NC, _BR, _D), lambda i: (0, i, 0)),
            pl.BlockSpec((_BR, _D), lambda i: (i, 0)),
            pl.BlockSpec((_BR, 1), lambda i: (i, 0)),
            pl.BlockSpec((1, _D), lambda i: (0, 0)),
            pl.BlockSpec((_D, n_classes), lambda i: (0, 0)),
            pl.BlockSpec((1, n_classes), lambda i: (0, 0)),
        ],
        out_specs=pl.BlockSpec((_BR, n_classes), lambda i: (i, 0)),
        out_shape=jax.ShapeDtypeStruct((_NPAD, n_classes), jnp.float32),
    )


def kernel(x, edge_index, W1, b1, W2, b2, W3, b3, Wl, bl):
    src = edge_index[0].astype(jnp.int32)
    dst = edge_index[1].astype(jnp.int32)
    pad_e = _EPAD - src.shape[0]
    src = jnp.pad(src, (0, pad_e), constant_values=_N).reshape(-1, _KB)
    dst = jnp.pad(dst, (0, pad_e), constant_values=_N).reshape(-1, _KB)
    xp = jnp.pad(x, ((0, _NPAD - x.shape[0]), (0, 0)))

    deg = _deg_kernel(dst)
    dis, g1 = _pre_call(deg, xp, W1)
    agg1 = _scatter_kernel(g1, src, dst)
    g2 = _mid_call(agg1, g1, dis, b1.reshape(1, -1), W2)
    agg2 = _scatter_kernel(g2, src, dst)
    g3 = _mid_call(agg2, g2, dis, b2.reshape(1, -1), W3)
    agg3 = _scatter_kernel(g3, src, dst)
    out = _make_fin(Wl.shape[1])(agg3, g3, dis, b3.reshape(1, -1), Wl,
                                 bl.reshape(1, -1))
    return out[:x.shape[0]]
